# baseline (speedup 1.0000x reference)
import functools

import jax
import jax.numpy as jnp
from jax import lax
from jax.experimental import pallas as pl
from jax.experimental.pallas import tpu as pltpu
from jax.experimental.pallas import tpu_sc as plsc

B = 10000
S = 10
D = 128
N_EMBED = 128

NC = 2
NS = 16
NW = NC * NS
CHUNK = 128

B_PAD = 10240
HALF = B_PAD // 2
SELF_PER_W = HALF // NW

SELF_CHUNKS = 3
SELF_SIZES = (16, 16, CHUNK)
SELF_OFFS = (0, 16, 32)

NEIGH_PER_W = SELF_PER_W * S
NEIGH_CHUNKS = 13
SMALL_NEIGH = 64

M_SELF = NW * SELF_PER_W
M_NEIGH = NW * NEIGH_PER_W

NBUF = 7
TOTAL_CHUNKS = SELF_CHUNKS + NEIGH_CHUNKS


def _neigh_off(k):
    return 0 if k == 0 else SMALL_NEIGH + (k - 1) * CHUNK


def _sc_gather_body(self_idx_hbm, neigh_idx_hbm, stab_hbm, ntab_hbm,
                    self_out_hbm, neigh_out_hbm,
                    idx_s_v, idx_n_v, rows_v, sem_g, sem_o):
    w = lax.axis_index("s") * NC + lax.axis_index("c")
    pltpu.sync_copy(self_idx_hbm.at[w], idx_s_v)
    pltpu.sync_copy(neigh_idx_hbm.at[w], idx_n_v)

    self_base = w * SELF_PER_W
    neigh_base = w * NEIGH_PER_W

    def size(c):
        if isinstance(c, int):
            if c < SELF_CHUNKS:
                return SELF_SIZES[c]
            if c == SELF_CHUNKS:
                return SMALL_NEIGH
        return CHUNK

    def buf(c):
        return rows_v.at[pl.ds(lax.rem(c, NBUF) * CHUNK, size(c))]

    def fire_gather(c):
        if isinstance(c, int) and c < SELF_CHUNKS:
            idx = idx_s_v.at[c, pl.ds(0, SELF_SIZES[c])]
            pltpu.async_copy(stab_hbm.at[idx], buf(c), sem_g.at[c % NBUF])
        elif isinstance(c, int) and c == SELF_CHUNKS:
            idx = idx_n_v.at[0, pl.ds(0, SMALL_NEIGH)]
            pltpu.async_copy(ntab_hbm.at[idx], buf(c), sem_g.at[c % NBUF])
        else:
            pltpu.async_copy(ntab_hbm.at[idx_n_v.at[c - SELF_CHUNKS]], buf(c),
                             sem_g.at[lax.rem(c, NBUF)])

    def wait_gather(c):
        pltpu.make_async_copy(neigh_out_hbm.at[pl.ds(neigh_base, size(c))],
                              buf(c), sem_g.at[lax.rem(c, NBUF)]).wait()

    def fire_out(c):
        if isinstance(c, int) and c < SELF_CHUNKS:
            dst = self_out_hbm.at[
                pl.ds(self_base + SELF_OFFS[c], SELF_SIZES[c])]
        elif isinstance(c, int) and c == SELF_CHUNKS:
            dst = neigh_out_hbm.at[pl.ds(neigh_base, SMALL_NEIGH)]
        else:
            dst = neigh_out_hbm.at[pl.ds(
                neigh_base + SMALL_NEIGH + (c - SELF_CHUNKS - 1) * CHUNK,
                CHUNK)]
        pltpu.async_copy(buf(c), dst, sem_o.at[lax.rem(c, NBUF)])

    def wait_out(c):
        dst = neigh_out_hbm.at[pl.ds(neigh_base, size(c))]
        pltpu.make_async_copy(buf(c), dst, sem_o.at[lax.rem(c, NBUF)]).wait()

    for c in range(NBUF):
        fire_gather(c)
    for c in range(SELF_CHUNKS):
        wait_gather(c)
        fire_out(c)
    for c in range(NBUF, SELF_CHUNKS + NBUF - 1):
        wait_out(c - NBUF)
        fire_gather(c)

    for c in range(SELF_CHUNKS, SELF_CHUNKS + 2):
        wait_out(c - 1)
        fire_gather(c + NBUF - 1)
        wait_gather(c)
        fire_out(c)

    @pl.loop(SELF_CHUNKS + 2, TOTAL_CHUNKS - NBUF + 1)
    def _steady(c):
        wait_out(c - 1)
        fire_gather(c + NBUF - 1)
        wait_gather(c)
        fire_out(c)

    for c in range(TOTAL_CHUNKS - NBUF + 1, TOTAL_CHUNKS):
        wait_out(c - 1)
        wait_gather(c)
        fire_out(c)
    wait_out(TOTAL_CHUNKS - 1)


@functools.cache
def _sc_gather():
    return pl.kernel(
        _sc_gather_body,
        out_type=(
            jax.ShapeDtypeStruct((M_SELF, D), jnp.float32),
            jax.ShapeDtypeStruct((M_NEIGH, D), jnp.float32),
        ),
        mesh=plsc.VectorSubcoreMesh(
            core_axis_name="c", subcore_axis_name="s",
            num_cores=NC, num_subcores=NS),
        scratch_types=[
            pltpu.VMEM((SELF_CHUNKS, CHUNK), jnp.int32),
            pltpu.VMEM((NEIGH_CHUNKS, CHUNK), jnp.int32),
            pltpu.VMEM((NBUF * CHUNK, D), jnp.float32),
            pltpu.SemaphoreType.DMA((NBUF,)),
            pltpu.SemaphoreType.DMA((NBUF,)),
        ],
    )


BLK = 1024
GRID = HALF // BLK


def _tc_dense_body(self_ref, neigh_ref, a1_ref, a2_ref, w1t_ref, w2t_ref,
                   out_ref):
    x = self_ref[...]
    a_self = jnp.dot(x, a1_ref[...], preferred_element_type=jnp.float32)

    logits = []
    for s in range(S):
        ns = neigh_ref[s]
        logits.append(
            jnp.dot(ns, a2_ref[...], preferred_element_type=jnp.float32)
            + a_self)
    lg = jnp.concatenate(logits, axis=1)
    wts = jnp.exp(jnp.maximum(lg, 0.0))
    wsum = jnp.sum(wts, axis=1, keepdims=True)

    agg = neigh_ref[0] * wts[:, 0:1]
    for s in range(1, S):
        agg = agg + neigh_ref[s] * wts[:, s:s + 1]
    agg = agg / wsum

    out = (jnp.dot(x, w1t_ref[...], preferred_element_type=jnp.float32)
           + jnp.dot(agg, w2t_ref[...], preferred_element_type=jnp.float32))
    out_ref[...] = jnp.maximum(out, 0.0)


def _stage_half(nodes_h, ni_h):
    nw_rows = nodes_h.reshape(NW, SELF_PER_W)
    self_idx = jnp.zeros((NW, SELF_CHUNKS, CHUNK), jnp.int32)
    for c in range(SELF_CHUNKS):
        self_idx = self_idx.at[:, c, :SELF_SIZES[c]].set(
            nw_rows[:, SELF_OFFS[c]:SELF_OFFS[c] + SELF_SIZES[c]])
    flat = ni_h.T.reshape(NW, NEIGH_PER_W)
    neigh_idx = jnp.zeros((NW, NEIGH_CHUNKS, CHUNK), jnp.int32)
    neigh_idx = neigh_idx.at[:, 0, :SMALL_NEIGH].set(flat[:, :SMALL_NEIGH])
    neigh_idx = neigh_idx.at[:, 1:, :].set(
        flat[:, SMALL_NEIGH:].reshape(NW, NEIGH_CHUNKS - 1, CHUNK))
    return self_idx, neigh_idx


@jax.jit
def kernel(nodes, neigh_index, self_feat_table, neigh_feat_table, weight,
           alpha):
    nodes_pad = jnp.zeros((B_PAD,), jnp.int32).at[:B].set(nodes)
    ni_pad = jnp.zeros((B_PAD, S), jnp.int32).at[:B].set(neigh_index)

    a1 = alpha[:D]
    a2 = alpha[D:]
    w1t = weight[:, :D].T
    w2t = weight[:, D:].T

    outs = []
    for h in range(2):
        nodes_h = nodes_pad[h * HALF:(h + 1) * HALF]
        ni_h = ni_pad[h * HALF:(h + 1) * HALF]
        self_idx, neigh_idx = _stage_half(nodes_h, ni_h)

        x, neigh_rows = _sc_gather()(
            self_idx, neigh_idx, self_feat_table, neigh_feat_table)
        y3 = neigh_rows.reshape(S, HALF, D)

        out_rows = HALF if h == 0 else B - HALF
        out = pl.pallas_call(
            _tc_dense_body,
            out_shape=jax.ShapeDtypeStruct((out_rows, N_EMBED), jnp.float32),
            grid=(GRID,),
            in_specs=[
                pl.BlockSpec((BLK, D), lambda i: (i, 0)),
                pl.BlockSpec((S, BLK, D), lambda i: (0, i, 0)),
                pl.BlockSpec((D, 1), lambda i: (0, 0)),
                pl.BlockSpec((D, 1), lambda i: (0, 0)),
                pl.BlockSpec((D, N_EMBED), lambda i: (0, 0)),
                pl.BlockSpec((D, N_EMBED), lambda i: (0, 0)),
            ],
            out_specs=pl.BlockSpec((BLK, N_EMBED), lambda i: (i, 0)),
        )(x, y3, a1, a2, w1t, w2t)
        outs.append(out)

    return jnp.concatenate(outs, axis=0)
